# Initial kernel scaffold; baseline (speedup 1.0000x reference)
import functools

import jax
import jax.numpy as jnp
from jax import lax
from jax.experimental import pallas as pl
from jax.experimental.pallas import tpu as pltpu
from jax.experimental.pallas import tpu_sc as plsc

D_IN = 16
D_OUT = 76
POS_COLS = (4, 5, 8, 9)
SCALE = 99.0
CHUNK = 128

_NC = 2
_NS = 16
_NW = _NC * _NS
_LANES = 16


def _sc_encode(n_rows: int):
    rows_per_w = n_rows // _NW
    mesh = plsc.VectorSubcoreMesh(core_axis_name="c", subcore_axis_name="s")

    scratch = [
        pltpu.VMEM((CHUNK, D_IN), jnp.float32),
        pltpu.VMEM((CHUNK, D_OUT), jnp.float32),
    ] + [pltpu.VMEM((CHUNK,), jnp.int32) for _ in POS_COLS] \
      + [pltpu.VMEM((CHUNK, D_IN), jnp.float32) for _ in POS_COLS]

    @functools.partial(
        pl.kernel,
        out_type=jax.ShapeDtypeStruct((n_rows, D_OUT), jnp.float32),
        mesh=mesh,
        scratch_types=scratch,
    )
    def body(x_hbm, emb_hbm, out_hbm, x_v, stage_v, i0, i1, i2, i3,
             g0, g1, g2, g3):
        idx_refs = (i0, i1, i2, i3)
        g_refs = (g0, g1, g2, g3)
        wid = lax.axis_index("s") * _NC + lax.axis_index("c")
        w_base = wid * rows_per_w
        lane_iota = lax.iota(jnp.int32, _LANES)

        @pl.loop(0, rows_per_w, step=CHUNK)
        def _chunk(off):
            base = w_base + off
            pltpu.sync_copy(x_hbm.at[pl.ds(base, CHUNK)], x_v)

            @pl.loop(0, CHUNK, step=_LANES)
            def _group(r):
                row_ids = lane_iota + r
                for k, col in enumerate(POS_COLS):
                    col_ids = jnp.full((_LANES,), col, jnp.int32)
                    vals = plsc.load_gather(x_v, [row_ids, col_ids])
                    idx_refs[k][pl.ds(r, _LANES)] = (vals * SCALE).astype(
                        jnp.int32)

            for k in range(4):
                pltpu.sync_copy(emb_hbm.at[idx_refs[k]], g_refs[k])

            pltpu.sync_copy(x_v.at[:, pl.ds(0, 4)], stage_v.at[:, pl.ds(0, 4)])
            pltpu.sync_copy(x_v.at[:, pl.ds(8, 8)],
                            stage_v.at[:, pl.ds(68, 8)])
            for k in range(4):
                pltpu.sync_copy(g_refs[k],
                                stage_v.at[:, pl.ds(4 + 16 * k, D_IN)])

            pltpu.sync_copy(stage_v, out_hbm.at[pl.ds(base, CHUNK)])

    return body


def kernel(x, embedding):
    b, l, d = x.shape
    xf = x.reshape(b * l, d)
    out = _sc_encode(b * l)(xf, embedding)
    return out.reshape(b, l, D_OUT)


# speedup vs baseline: 6.2145x; 6.2145x over previous
import dataclasses
import functools

import jax
import jax.numpy as jnp
from jax import lax
from jax.experimental import pallas as pl
from jax.experimental.pallas import tpu as pltpu
from jax.experimental.pallas import tpu_sc as plsc

D_IN = 16
D_OUT = 76
POS_COLS = (4, 5, 8, 9)
SCALE = 99.0
CHUNK = 128

_NC = 2
_NS = 16
_NW = _NC * _NS
_LANES = 16


def _sc_encode(n_rows: int):
    rows_per_w = n_rows // _NW
    mesh = plsc.VectorSubcoreMesh(core_axis_name="c", subcore_axis_name="s")

    scratch = [
        pltpu.VMEM((CHUNK, D_IN), jnp.float32),
        pltpu.VMEM((CHUNK, D_OUT), jnp.float32),
    ] + [pltpu.VMEM((CHUNK,), jnp.int32) for _ in POS_COLS] \
      + [pltpu.VMEM((CHUNK, D_IN), jnp.float32) for _ in POS_COLS]

    cp = pltpu.CompilerParams(needs_layout_passes=False,
                              use_tc_tiling_on_sc=False)

    @functools.partial(
        pl.kernel,
        out_type=jax.ShapeDtypeStruct((n_rows, D_OUT), jnp.float32),
        mesh=mesh,
        scratch_types=scratch,
        compiler_params=cp,
    )
    def body(x_hbm, emb_hbm, out_hbm, x_v, stage_v, i0, i1, i2, i3,
             g0, g1, g2, g3):
        idx_refs = (i0, i1, i2, i3)
        g_refs = (g0, g1, g2, g3)
        wid = lax.axis_index("s") * _NC + lax.axis_index("c")
        w_base = wid * rows_per_w
        lane_iota = lax.iota(jnp.int32, _LANES)
        pass_map = jnp.where(lane_iota >= 8, lane_iota + 60, lane_iota)
        pass_mask = lane_iota // 4 != 1

        @pl.loop(0, rows_per_w, step=CHUNK)
        def _chunk(off):
            base = w_base + off
            pltpu.sync_copy(x_hbm.at[pl.ds(base, CHUNK)], x_v)

            @pl.loop(0, CHUNK, step=_LANES)
            def _group(r):
                row_ids = lane_iota + r
                for k, col in enumerate(POS_COLS):
                    col_ids = jnp.full((_LANES,), col, jnp.int32)
                    vals = plsc.load_gather(x_v, [row_ids, col_ids])
                    idx_refs[k][pl.ds(r, _LANES)] = (vals * SCALE).astype(
                        jnp.int32)

            for k in range(4):
                pltpu.sync_copy(emb_hbm.at[idx_refs[k]], g_refs[k])

            @pl.loop(0, CHUNK)
            def _row(r):
                r_vec = jnp.full((_LANES,), r, jnp.int32)
                for k in range(4):
                    plsc.store_scatter(stage_v,
                                       [r_vec, lane_iota + (4 + 16 * k)],
                                       g_refs[k][r])
                plsc.store_scatter(stage_v, [r_vec, pass_map], x_v[r],
                                   mask=pass_mask)

            pltpu.sync_copy(stage_v, out_hbm.at[pl.ds(base, CHUNK)])

    return body


def kernel(x, embedding):
    b, l, d = x.shape
    xf = x.reshape(b * l, d)
    out = _sc_encode(b * l)(xf, embedding)
    return out.reshape(b, l, D_OUT)


# speedup vs baseline: 7.2621x; 1.1686x over previous
import functools

import jax
import jax.numpy as jnp
from jax import lax
from jax.experimental import pallas as pl
from jax.experimental.pallas import tpu as pltpu
from jax.experimental.pallas import tpu_sc as plsc

D_IN = 16
D_OUT = 76
N_EMB = 100
POS_COLS = (4, 5, 8, 9)
EMB_OFFS = (4, 20, 36, 52)
SCALE = 99.0
CHUNK = 128

_NC = 2
_NS = 16
_NW = _NC * _NS
_LANES = 16


def _sc_encode(n_rows: int):
    rows_per_w = n_rows // _NW
    mesh = plsc.VectorSubcoreMesh(core_axis_name="c", subcore_axis_name="s")

    scratch = [
        pltpu.VMEM((CHUNK, D_IN), jnp.float32),
        pltpu.VMEM((CHUNK, D_OUT), jnp.float32),
        pltpu.VMEM((N_EMB, D_IN), jnp.float32),
    ]

    cp = pltpu.CompilerParams(needs_layout_passes=False,
                              use_tc_tiling_on_sc=False)

    @functools.partial(
        pl.kernel,
        out_type=jax.ShapeDtypeStruct((n_rows, D_OUT), jnp.float32),
        mesh=mesh,
        scratch_types=scratch,
        compiler_params=cp,
    )
    def body(x_hbm, emb_hbm, out_hbm, x_v, stage_v, emb_v):
        wid = lax.axis_index("s") * _NC + lax.axis_index("c")
        w_base = wid * rows_per_w
        lane_iota = lax.iota(jnp.int32, _LANES)

        pltpu.sync_copy(emb_hbm, emb_v)

        def fullv(c):
            return jnp.full((_LANES,), c, jnp.int32)

        @pl.loop(0, rows_per_w, step=CHUNK)
        def _chunk(off):
            base = w_base + off
            pltpu.sync_copy(x_hbm.at[pl.ds(base, CHUNK)], x_v)

            @pl.loop(0, CHUNK, step=_LANES)
            def _group(r):
                rows = lane_iota + r
                qs = []
                for col in POS_COLS:
                    vals = plsc.load_gather(x_v, [rows, fullv(col)])
                    qs.append((vals * SCALE).astype(jnp.int32))
                for c in range(4):
                    v = plsc.load_gather(x_v, [rows, fullv(c)])
                    plsc.store_scatter(stage_v, [rows, fullv(c)], v)
                for c in range(8, 16):
                    v = plsc.load_gather(x_v, [rows, fullv(c)])
                    plsc.store_scatter(stage_v, [rows, fullv(60 + c)], v)
                for k in range(4):
                    for j in range(D_IN):
                        v = plsc.load_gather(emb_v, [qs[k], fullv(j)])
                        plsc.store_scatter(
                            stage_v, [rows, fullv(EMB_OFFS[k] + j)], v)

            pltpu.sync_copy(stage_v, out_hbm.at[pl.ds(base, CHUNK)])

    return body


def kernel(x, embedding):
    b, l, d = x.shape
    xf = x.reshape(b * l, d)
    out = _sc_encode(b * l)(xf, embedding)
    return out.reshape(b, l, D_OUT)
